# baseline (speedup 1.0000x reference)
import functools

import jax
import jax.numpy as jnp
from jax import lax
from jax.experimental import pallas as pl
from jax.experimental.pallas import tpu as pltpu
from jax.experimental.pallas import tpu_sc as plsc

N, C, V = 320000, 128, 10000
B = 1600
NB = N // B
BW = B + 8
FASTW = 136
ACC_ROWS = V + BW + 8

CH = 128
SCH = 256
NCHUNK = N // SCH


def _seg_body(x_ref, ix_ref, ixp_ref, wfg_ref, bfg_ref, wht_ref, bh_ref,
              tri_ref, y2_ref, seg_ref, acc_ref, s_smem):
    i = pl.program_id(0)

    @pl.when(i == 0)
    def _init():
        s_smem[0] = 0
        acc_ref[...] = jnp.zeros_like(acc_ref)

    t = jax.lax.dot_general(x_ref[...], wfg_ref[...],
                            (((1,), (0,)), ((), ())),
                            preferred_element_type=jnp.float32)
    t = t + bfg_ref[...]
    fx = t[:, :C]
    e = jnp.exp(t[:, C:])
    contrib = jnp.concatenate([fx * e, e], axis=1)

    ixr = ix_ref[0]
    ixpr = ixp_ref[0]
    flags = (ixr != ixpr).astype(jnp.float32)

    csum = jax.lax.dot_general(flags, tri_ref[...], (((1,), (0,)), ((), ())),
                               preferred_element_type=jnp.float32)
    total = jnp.sum(flags).astype(jnp.int32)

    s_base = s_smem[0]
    seg_row = s_base + csum.astype(jnp.int32)
    seg_ref[0] = seg_row
    s_smem[0] = s_base + total

    s_al = (s_base // 8) * 8
    shift = (s_base - s_al).astype(jnp.float32)
    rel = (csum + shift).astype(jnp.int32)

    onehot = (lax.broadcasted_iota(jnp.int32, (FASTW, B), 0) ==
              jnp.broadcast_to(rel, (FASTW, B))).astype(jnp.float32)
    partial = jax.lax.dot_general(onehot, contrib,
                                  (((1,), (0,)), ((), ())),
                                  preferred_element_type=jnp.float32)
    acc_ref[pl.ds(s_al, FASTW), :] = (
        acc_ref[pl.ds(s_al, FASTW), :] + partial)

    @pl.when(total >= FASTW - 8)
    def _residual():
        oh2 = ((lax.broadcasted_iota(jnp.int32, (BW, B), 0) ==
                jnp.broadcast_to(rel, (BW, B))) &
               jnp.broadcast_to(rel >= FASTW, (BW, B))).astype(jnp.float32)
        p2 = jax.lax.dot_general(oh2, contrib, (((1,), (0,)), ((), ())),
                                 preferred_element_type=jnp.float32)
        acc_ref[pl.ds(s_al, BW), :] = acc_ref[pl.ds(s_al, BW), :] + p2

    @pl.when(i == NB - 1)
    def _finalize():
        num = acc_ref[:V, :C]
        den = acc_ref[:V, C:]
        y = num / jnp.where(den > 0, den, 1.0)
        y2 = jax.lax.dot_general(y, wht_ref[...], (((1,), (0,)), ((), ())),
                                 preferred_element_type=jnp.float32)
        y2_ref[...] = y2 + bh_ref[...]


def _seg_call(x, ix3, ixp3, Wfg, bfg, WhT, bh2, tri, interpret=False):
    return pl.pallas_call(
        _seg_body,
        grid=(NB,),
        in_specs=[
            pl.BlockSpec((B, C), lambda i: (i, 0)),
            pl.BlockSpec((1, 1, B), lambda i: (i, 0, 0)),
            pl.BlockSpec((1, 1, B), lambda i: (i, 0, 0)),
            pl.BlockSpec((C, 2 * C), lambda i: (0, 0)),
            pl.BlockSpec((1, 2 * C), lambda i: (0, 0)),
            pl.BlockSpec((C, C), lambda i: (0, 0)),
            pl.BlockSpec((1, C), lambda i: (0, 0)),
            pl.BlockSpec((B, B), lambda i: (0, 0)),
        ],
        out_specs=[
            pl.BlockSpec((V, C), lambda i: (0, 0)),
            pl.BlockSpec((1, 1, B), lambda i: (i, 0, 0)),
        ],
        out_shape=[
            jax.ShapeDtypeStruct((V, C), jnp.float32),
            jax.ShapeDtypeStruct((NB, 1, B), jnp.int32),
        ],
        scratch_shapes=[
            pltpu.VMEM((ACC_ROWS, 2 * C), jnp.float32),
            pltpu.SMEM((1,), jnp.int32),
        ],
        compiler_params=pltpu.CompilerParams(
            dimension_semantics=("arbitrary",)),
        interpret=interpret,
    )(x, ix3, ixp3, Wfg, bfg, WhT, bh2, tri)


_NC, _NS = 2, 16
_NW = _NC * _NS
_TRIPS = (NCHUNK + _NW - 1) // _NW
_TPAD = ((_TRIPS + 2) // 3) * 3


@functools.cache
def _sc_gather_kernel():
    mesh = plsc.VectorSubcoreMesh(
        core_axis_name="c", subcore_axis_name="s", num_cores=_NC,
        num_subcores=_NS)

    @functools.partial(
        pl.kernel,
        mesh=mesh,
        out_type=jax.ShapeDtypeStruct((N, C), jnp.float32),
        scratch_types=[
            pltpu.VMEM((SCH,), jnp.int32),
            pltpu.VMEM((SCH,), jnp.int32),
            pltpu.VMEM((SCH,), jnp.int32),
            pltpu.VMEM((SCH, C), jnp.float32),
            pltpu.VMEM((SCH, C), jnp.float32),
            pltpu.VMEM((SCH, C), jnp.float32),
            pltpu.SemaphoreType.DMA,
            pltpu.SemaphoreType.DMA,
            pltpu.SemaphoreType.DMA,
            pltpu.SemaphoreType.DMA,
            pltpu.SemaphoreType.DMA,
            pltpu.SemaphoreType.DMA,
        ],
    )
    def _sc_gather(y_hbm, seg_hbm, out_hbm, idx0, idx1, idx2,
                   rows0, rows1, rows2, g0, g1, g2, s0, s1, s2):
        wid = lax.axis_index("s") * _NC + lax.axis_index("c")
        wbase = wid * _TRIPS
        bufs = ((idx0, rows0, g0, s0), (idx1, rows1, g1, s1),
                (idx2, rows2, g2, s2))

        def cbase(j):
            return jnp.minimum(wbase + j, NCHUNK - 1) * SCH

        def load_gather(j, b):
            idx_v, rows_v, gsem, _ = bufs[b]
            pltpu.sync_copy(seg_hbm.at[pl.ds(cbase(j), SCH)], idx_v)
            pltpu.async_copy(y_hbm.at[idx_v.at[pl.ds(0, CH)]],
                             rows_v.at[pl.ds(0, CH)], gsem)
            pltpu.async_copy(y_hbm.at[idx_v.at[pl.ds(CH, CH)]],
                             rows_v.at[pl.ds(CH, CH)], gsem)

        def wait_gather(b):
            idx_v, rows_v, gsem, _ = bufs[b]
            pltpu.make_async_copy(y_hbm.at[idx_v.at[pl.ds(0, CH)]],
                                  rows_v.at[pl.ds(0, CH)], gsem).wait()
            pltpu.make_async_copy(y_hbm.at[idx_v.at[pl.ds(CH, CH)]],
                                  rows_v.at[pl.ds(CH, CH)], gsem).wait()

        def scatter(j, b):
            _, rows_v, _, ssem = bufs[b]
            pltpu.async_copy(rows_v, out_hbm.at[pl.ds(cbase(j), SCH)], ssem)

        def wait_scatter(j, b):
            _, rows_v, _, ssem = bufs[b]
            pltpu.make_async_copy(
                rows_v, out_hbm.at[pl.ds(cbase(j), SCH)], ssem).wait()

        load_gather(0, 0)
        load_gather(1, 1)
        wait_gather(0)
        scatter(0, 0)
        load_gather(2, 2)
        wait_gather(1)
        scatter(1, 1)

        def step(m, carry):
            for u in range(3):
                j = 3 * m + 3 + u
                wait_scatter(j - 3, u)
                load_gather(j, u)
                wait_gather((u + 2) % 3)
                scatter(j - 1, (u + 2) % 3)
            return carry

        lax.fori_loop(0, (_TPAD - 3) // 3, step, 0)

        jl = _TPAD - 1
        wait_gather(jl % 3)
        scatter(jl, jl % 3)
        wait_scatter(jl - 2, (jl - 2) % 3)
        wait_scatter(jl - 1, (jl - 1) % 3)
        wait_scatter(jl, jl % 3)

    return _sc_gather


def kernel(x, ix, Wf, bf, Wg, bg, Wh, bh):
    ix = ix.astype(jnp.int32)
    ixp = jnp.concatenate([ix[:1], ix[:-1]])
    Wfg = jnp.concatenate([Wf.T, Wg.T], axis=1)
    bfg = jnp.concatenate([bf, bg]).reshape(1, 2 * C)
    ix3 = ix.reshape(NB, 1, B)
    ixp3 = ixp.reshape(NB, 1, B)
    io = jnp.arange(B, dtype=jnp.int32)
    tri = (io[:, None] <= io[None, :]).astype(jnp.float32)
    y2, seg3 = _seg_call(x, ix3, ixp3, Wfg, bfg, Wh.T, bh.reshape(1, C), tri)
    seg = seg3.reshape(N)
    return _sc_gather_kernel()(y2, seg)


# speedup vs baseline: 1.0169x; 1.0169x over previous
import functools

import jax
import jax.numpy as jnp
from jax import lax
from jax.experimental import pallas as pl
from jax.experimental.pallas import tpu as pltpu
from jax.experimental.pallas import tpu_sc as plsc

N, C, V = 320000, 128, 10000
B = 1600
NB = N // B
BW = B + 8
FASTW = 136
ACC_ROWS = V + BW + 8

CH = 128
SCH = 256
NCHUNK = N // SCH


def _seg_body(x_ref, ix_ref, ixp_ref, wfg_ref, bfg_ref, wht_ref, bh_ref,
              tri_ref, y2_ref, seg_ref, acc_ref, s_smem):
    i = pl.program_id(0)

    @pl.when(i == 0)
    def _init():
        s_smem[0] = 0
        acc_ref[...] = jnp.zeros_like(acc_ref)

    t = jax.lax.dot_general(x_ref[...], wfg_ref[...],
                            (((1,), (0,)), ((), ())),
                            preferred_element_type=jnp.float32)
    t = t + bfg_ref[...]
    fx = t[:, :C]
    e = jnp.exp(t[:, C:])
    contrib = jnp.concatenate([fx * e, e], axis=1)

    ixr = ix_ref[0]
    ixpr = ixp_ref[0]
    flags = (ixr != ixpr).astype(jnp.float32)

    csum = jax.lax.dot_general(flags, tri_ref[...], (((1,), (0,)), ((), ())),
                               preferred_element_type=jnp.float32)
    total = jnp.sum(flags).astype(jnp.int32)

    s_base = s_smem[0]
    seg_row = s_base + csum.astype(jnp.int32)
    seg_ref[0] = seg_row
    s_smem[0] = s_base + total

    s_al = (s_base // 8) * 8
    shift = (s_base - s_al).astype(jnp.float32)
    rel = (csum + shift).astype(jnp.int32)

    onehot = (lax.broadcasted_iota(jnp.int32, (FASTW, B), 0) ==
              jnp.broadcast_to(rel, (FASTW, B))).astype(jnp.float32)
    partial = jax.lax.dot_general(onehot, contrib,
                                  (((1,), (0,)), ((), ())),
                                  preferred_element_type=jnp.float32)
    acc_ref[pl.ds(s_al, FASTW), :] = (
        acc_ref[pl.ds(s_al, FASTW), :] + partial)

    @pl.when(total >= FASTW - 8)
    def _residual():
        oh2 = ((lax.broadcasted_iota(jnp.int32, (BW, B), 0) ==
                jnp.broadcast_to(rel, (BW, B))) &
               jnp.broadcast_to(rel >= FASTW, (BW, B))).astype(jnp.float32)
        p2 = jax.lax.dot_general(oh2, contrib, (((1,), (0,)), ((), ())),
                                 preferred_element_type=jnp.float32)
        acc_ref[pl.ds(s_al, BW), :] = acc_ref[pl.ds(s_al, BW), :] + p2

    @pl.when(i == NB - 1)
    def _finalize():
        num = acc_ref[:V, :C]
        den = acc_ref[:V, C:]
        y = num / jnp.where(den > 0, den, 1.0)
        y2 = jax.lax.dot_general(y, wht_ref[...], (((1,), (0,)), ((), ())),
                                 preferred_element_type=jnp.float32)
        y2_ref[...] = y2 + bh_ref[...]


def _seg_call(x, ix3, ixp3, Wfg, bfg, WhT, bh2, tri):
    return pl.pallas_call(
        _seg_body,
        grid=(NB,),
        in_specs=[
            pl.BlockSpec((B, C), lambda i: (i, 0)),
            pl.BlockSpec((1, 1, B), lambda i: (i, 0, 0)),
            pl.BlockSpec((1, 1, B), lambda i: (i, 0, 0)),
            pl.BlockSpec((C, 2 * C), lambda i: (0, 0)),
            pl.BlockSpec((1, 2 * C), lambda i: (0, 0)),
            pl.BlockSpec((C, C), lambda i: (0, 0)),
            pl.BlockSpec((1, C), lambda i: (0, 0)),
            pl.BlockSpec((B, B), lambda i: (0, 0)),
        ],
        out_specs=[
            pl.BlockSpec((V, C), lambda i: (0, 0)),
            pl.BlockSpec((1, 1, B), lambda i: (i, 0, 0)),
        ],
        out_shape=[
            jax.ShapeDtypeStruct((V, C), jnp.float32),
            jax.ShapeDtypeStruct((NB, 1, B), jnp.int32),
        ],
        scratch_shapes=[
            pltpu.VMEM((ACC_ROWS, 2 * C), jnp.float32),
            pltpu.SMEM((1,), jnp.int32),
        ],
        compiler_params=pltpu.CompilerParams(
            dimension_semantics=("arbitrary",)),
    )(x, ix3, ixp3, Wfg, bfg, WhT, bh2, tri)


_NC, _NS = 2, 16
_NW = _NC * _NS
_TRIPS = (NCHUNK + _NW - 1) // _NW


@functools.cache
def _sc_gather_kernel():
    mesh = plsc.VectorSubcoreMesh(
        core_axis_name="c", subcore_axis_name="s", num_cores=_NC,
        num_subcores=_NS)

    @functools.partial(
        pl.kernel,
        mesh=mesh,
        out_type=jax.ShapeDtypeStruct((N, C), jnp.float32),
        scratch_types=[
            pltpu.VMEM((SCH,), jnp.int32),
            pltpu.VMEM((SCH,), jnp.int32),
            pltpu.VMEM((SCH, C), jnp.float32),
            pltpu.VMEM((SCH, C), jnp.float32),
            pltpu.SemaphoreType.DMA,
            pltpu.SemaphoreType.DMA,
            pltpu.SemaphoreType.DMA,
            pltpu.SemaphoreType.DMA,
        ],
    )
    def _sc_gather(y_hbm, seg_hbm, out_hbm, idx0, idx1, rows0, rows1,
                   g0, g1, s0, s1):
        wid = lax.axis_index("s") * _NC + lax.axis_index("c")
        wbase = wid * _TRIPS

        def cbase(j):
            return jnp.minimum(wbase + j, NCHUNK - 1) * SCH

        def load_gather(j, idx_v, rows_v, gsem):
            pltpu.sync_copy(seg_hbm.at[pl.ds(cbase(j), SCH)], idx_v)
            pltpu.async_copy(y_hbm.at[idx_v.at[pl.ds(0, CH)]],
                             rows_v.at[pl.ds(0, CH)], gsem)
            pltpu.async_copy(y_hbm.at[idx_v.at[pl.ds(CH, CH)]],
                             rows_v.at[pl.ds(CH, CH)], gsem)

        def wait_gather(idx_v, rows_v, gsem):
            pltpu.make_async_copy(y_hbm.at[idx_v.at[pl.ds(0, CH)]],
                                  rows_v.at[pl.ds(0, CH)], gsem).wait()
            pltpu.make_async_copy(y_hbm.at[idx_v.at[pl.ds(CH, CH)]],
                                  rows_v.at[pl.ds(CH, CH)], gsem).wait()

        def scatter(j, rows_v, ssem):
            pltpu.async_copy(rows_v, out_hbm.at[pl.ds(cbase(j), SCH)], ssem)

        def wait_scatter(j, rows_v, ssem):
            pltpu.make_async_copy(
                rows_v, out_hbm.at[pl.ds(cbase(j), SCH)], ssem).wait()

        load_gather(0, idx0, rows0, g0)
        load_gather(1, idx1, rows1, g1)
        wait_gather(idx0, rows0, g0)
        scatter(0, rows0, s0)

        def step(k, carry):
            ja = 2 * k + 2
            jb = 2 * k + 3
            wait_scatter(ja - 2, rows0, s0)
            load_gather(ja, idx0, rows0, g0)
            wait_gather(idx1, rows1, g1)
            scatter(jb - 2, rows1, s1)
            wait_scatter(jb - 2, rows1, s1)
            load_gather(jb, idx1, rows1, g1)
            wait_gather(idx0, rows0, g0)
            scatter(ja, rows0, s0)
            return carry

        lax.fori_loop(0, (_TRIPS - 2) // 2, step, 0)

        jl = _TRIPS - 1
        wait_gather(idx1, rows1, g1)
        scatter(jl, rows1, s1)
        wait_scatter(jl - 1, rows0, s0)
        wait_scatter(jl, rows1, s1)

    return _sc_gather


def kernel(x, ix, Wf, bf, Wg, bg, Wh, bh):
    ix = ix.astype(jnp.int32)
    ixp = jnp.concatenate([ix[:1], ix[:-1]])
    Wfg = jnp.concatenate([Wf.T, Wg.T], axis=1)
    bfg = jnp.concatenate([bf, bg]).reshape(1, 2 * C)
    ix3 = ix.reshape(NB, 1, B)
    ixp3 = ixp.reshape(NB, 1, B)
    io = jnp.arange(B, dtype=jnp.int32)
    tri = (io[:, None] <= io[None, :]).astype(jnp.float32)
    y2, seg3 = _seg_call(x, ix3, ixp3, Wfg, bfg, Wh.T, bh.reshape(1, C), tri)
    seg = seg3.reshape(N)
    return _sc_gather_kernel()(y2, seg)
